# Initial kernel scaffold; baseline (speedup 1.0000x reference)
import jax
import jax.numpy as jnp
from jax.experimental import pallas as pl


def kernel(x_bs, tok_weight, pos_weight):
    raise NotImplementedError("write your pallas kernel here")


# speedup vs baseline: 1.3207x; 1.3207x over previous
import math
import functools

import jax
import jax.numpy as jnp
from jax import lax
from jax.experimental import pallas as pl
from jax.experimental.pallas import tpu as pltpu
from jax.experimental.pallas import tpu_sc as plsc

BATCH = 4
SEQ_LEN = 2048
EMB = 128
TOTAL = BATCH * SEQ_LEN
NUM_WORKERS = 32
PER_W = TOTAL // NUM_WORKERS
CHUNK = 128
N_CHUNKS = PER_W // CHUNK
LANES = 16
SCALE = math.sqrt(EMB)


def _body(x_hbm, tok_hbm, pos_hbm, out_hbm, idx_v, rows_v, pos_v, sem):
    c = lax.axis_index("c")
    s = lax.axis_index("s")
    wid = s * 2 + c
    base = wid * PER_W
    s_base = lax.rem(base, SEQ_LEN)

    pltpu.sync_copy(x_hbm.at[pl.ds(wid * N_CHUNKS, N_CHUNKS)], idx_v)

    copies = []
    for j in range(N_CHUNKS):
        copies.append(
            pltpu.async_copy(
                tok_hbm.at[idx_v.at[j]],
                rows_v.at[pl.ds(j * CHUNK, CHUNK)],
                sem,
            )
        )
    pltpu.sync_copy(pos_hbm.at[pl.ds(s_base, PER_W)], pos_v)
    for cp in copies:
        cp.wait()

    def row(i, carry):
        for k in range(EMB // LANES):
            sl = pl.ds(k * LANES, LANES)
            t = rows_v[i, sl]
            p = pos_v[i, sl]
            rows_v[i, sl] = (t + p) * SCALE
        return carry

    lax.fori_loop(0, PER_W, row, 0)

    pltpu.sync_copy(rows_v, out_hbm.at[pl.ds(base, PER_W)])


def kernel(x_bs, tok_weight, pos_weight):
    x_flat = x_bs.reshape(TOTAL // CHUNK, CHUNK).astype(jnp.int32)

    mesh = plsc.VectorSubcoreMesh(core_axis_name="c", subcore_axis_name="s")
    run = functools.partial(
        pl.kernel,
        mesh=mesh,
        out_type=jax.ShapeDtypeStruct((TOTAL, EMB), jnp.float32),
        scratch_types=[
            pltpu.VMEM((N_CHUNKS, CHUNK), jnp.int32),
            pltpu.VMEM((PER_W, EMB), jnp.float32),
            pltpu.VMEM((PER_W, EMB), jnp.float32),
            pltpu.SemaphoreType.DMA,
        ],
    )(_body)

    out = run(x_flat, tok_weight, pos_weight)
    return out.reshape(BATCH, SEQ_LEN, EMB)


# speedup vs baseline: 1.4181x; 1.0737x over previous
import math
import functools

import jax
import jax.numpy as jnp
from jax import lax
from jax.experimental import pallas as pl
from jax.experimental.pallas import tpu as pltpu
from jax.experimental.pallas import tpu_sc as plsc

BATCH = 4
SEQ_LEN = 2048
EMB = 128
NUM_WORKERS = 32
S_PER_W = SEQ_LEN // NUM_WORKERS
ROWS_PER_W = BATCH * S_PER_W
LANES = 16
KREG = EMB // LANES
SCALE = math.sqrt(EMB)


def _body(x_hbm, tok_hbm, pos_hbm, out_hbm, idx_v, rows_v, pos_v, sem):
    c = lax.axis_index("c")
    s = lax.axis_index("s")
    wid = s * 2 + c
    s_base = wid * S_PER_W

    idx_copies = [
        pltpu.async_copy(
            x_hbm.at[pl.ds(b * SEQ_LEN + s_base, S_PER_W)],
            idx_v.at[b],
            sem,
        )
        for b in range(BATCH)
    ]
    for cp in idx_copies:
        cp.wait()

    gathers = [
        pltpu.async_copy(
            tok_hbm.at[idx_v.at[b]],
            rows_v.at[pl.ds(b * S_PER_W, S_PER_W)],
            sem,
        )
        for b in range(BATCH)
    ]
    pltpu.sync_copy(pos_hbm.at[pl.ds(s_base, S_PER_W)], pos_v)
    for cp in gathers:
        cp.wait()

    def srow(i, carry):
        p = [pos_v[i, pl.ds(k * LANES, LANES)] for k in range(KREG)]
        for b in range(BATCH):
            r = b * S_PER_W + i
            for k in range(KREG):
                sl = pl.ds(k * LANES, LANES)
                rows_v[r, sl] = (rows_v[r, sl] + p[k]) * SCALE
        return carry

    lax.fori_loop(0, S_PER_W, srow, 0)

    out_copies = [
        pltpu.async_copy(
            rows_v.at[pl.ds(b * S_PER_W, S_PER_W)],
            out_hbm.at[b, pl.ds(s_base, S_PER_W)],
            sem,
        )
        for b in range(BATCH)
    ]
    for cp in out_copies:
        cp.wait()


def kernel(x_bs, tok_weight, pos_weight):
    x_flat = x_bs.reshape(BATCH * SEQ_LEN)

    mesh = plsc.VectorSubcoreMesh(core_axis_name="c", subcore_axis_name="s")
    run = functools.partial(
        pl.kernel,
        mesh=mesh,
        out_type=jax.ShapeDtypeStruct((BATCH, SEQ_LEN, EMB), jnp.float32),
        scratch_types=[
            pltpu.VMEM((BATCH, S_PER_W), jnp.int32),
            pltpu.VMEM((ROWS_PER_W, EMB), jnp.float32),
            pltpu.VMEM((S_PER_W, EMB), jnp.float32),
            pltpu.SemaphoreType.DMA,
        ],
    )(_body)

    return run(x_flat, tok_weight, pos_weight)


# speedup vs baseline: 1.4410x; 1.0162x over previous
import math
import functools

import jax
import jax.numpy as jnp
from jax import lax
from jax.experimental import pallas as pl
from jax.experimental.pallas import tpu as pltpu
from jax.experimental.pallas import tpu_sc as plsc

BATCH = 4
SEQ_LEN = 2048
EMB = 128
NUM_WORKERS = 32
S_PER_W = SEQ_LEN // NUM_WORKERS
ROWS_PER_W = BATCH * S_PER_W
LANES = 16
KREG = EMB // LANES
SCALE = math.sqrt(EMB)


def _body(x_hbm, tok_hbm, pos_hbm, out_hbm,
          idx_v, rows_v, pos_v, isem, gsem, osem, psem):
    c = lax.axis_index("c")
    s = lax.axis_index("s")
    wid = s * 2 + c
    s_base = wid * S_PER_W

    idx_copies = [
        pltpu.async_copy(
            x_hbm.at[b, pl.ds(s_base, S_PER_W)],
            idx_v.at[b],
            isem.at[b],
        )
        for b in range(BATCH)
    ]
    pos_copy = pltpu.async_copy(pos_hbm.at[pl.ds(s_base, S_PER_W)], pos_v, psem)
    gathers = []
    for b in range(BATCH):
        idx_copies[b].wait()
        gathers.append(
            pltpu.async_copy(
                tok_hbm.at[idx_v.at[b]],
                rows_v.at[pl.ds(b * S_PER_W, S_PER_W)],
                gsem.at[b],
            )
        )
    pos_copy.wait()

    out_copies = []
    for pair in range(BATCH // 2):
        b0 = pair * 2
        gathers[b0].wait()
        gathers[b0 + 1].wait()

        def srow(i, carry):
            p = [pos_v[i, pl.ds(k * LANES, LANES)] for k in range(KREG)]
            for b in (b0, b0 + 1):
                r = b * S_PER_W + i
                for k in range(KREG):
                    sl = pl.ds(k * LANES, LANES)
                    rows_v[r, sl] = (rows_v[r, sl] + p[k]) * SCALE
            return carry

        lax.fori_loop(0, S_PER_W, srow, 0)

        for b in (b0, b0 + 1):
            out_copies.append(
                pltpu.async_copy(
                    rows_v.at[pl.ds(b * S_PER_W, S_PER_W)],
                    out_hbm.at[b, pl.ds(s_base, S_PER_W)],
                    osem.at[b],
                )
            )
    for cp in out_copies:
        cp.wait()


def kernel(x_bs, tok_weight, pos_weight):
    mesh = plsc.VectorSubcoreMesh(core_axis_name="c", subcore_axis_name="s")
    run = functools.partial(
        pl.kernel,
        mesh=mesh,
        out_type=jax.ShapeDtypeStruct((BATCH, SEQ_LEN, EMB), jnp.float32),
        scratch_types=[
            pltpu.VMEM((BATCH, S_PER_W), jnp.int32),
            pltpu.VMEM((ROWS_PER_W, EMB), jnp.float32),
            pltpu.VMEM((S_PER_W, EMB), jnp.float32),
            pltpu.SemaphoreType.DMA((BATCH,)),
            pltpu.SemaphoreType.DMA((BATCH,)),
            pltpu.SemaphoreType.DMA((BATCH,)),
            pltpu.SemaphoreType.DMA,
        ],
    )(_body)

    return run(x_bs, tok_weight, pos_weight)


# speedup vs baseline: 1.4624x; 1.0149x over previous
import math
import functools

import jax
import jax.numpy as jnp
from jax import lax
from jax.experimental import pallas as pl
from jax.experimental.pallas import tpu as pltpu
from jax.experimental.pallas import tpu_sc as plsc

BATCH = 4
SEQ_LEN = 2048
EMB = 128
NUM_WORKERS = 32
S_PER_W = SEQ_LEN // NUM_WORKERS
ROWS_PER_W = BATCH * S_PER_W
LANES = 16
KREG = EMB // LANES
SCALE = math.sqrt(EMB)


S_CHUNK = 32
N_SC = S_PER_W // S_CHUNK


def _body(x_hbm, tok_hbm, pos_hbm, out_hbm,
          idx_v, rows_v, pos_v, isem, gsem, osem, psem):
    c = lax.axis_index("c")
    s = lax.axis_index("s")
    wid = s * 2 + c
    s_base = wid * S_PER_W

    idx_copies = [
        pltpu.async_copy(
            x_hbm.at[b, pl.ds(s_base, S_PER_W)],
            idx_v.at[b],
            isem.at[b],
        )
        for b in range(BATCH)
    ]
    pos_copy = pltpu.async_copy(pos_hbm.at[pl.ds(s_base, S_PER_W)], pos_v, psem)

    gathers = {}

    def fire_gather(b, sc):
        off = b * S_PER_W + sc * S_CHUNK
        gathers[(b, sc)] = pltpu.async_copy(
            tok_hbm.at[idx_v.at[b, pl.ds(sc * S_CHUNK, S_CHUNK)]],
            rows_v.at[pl.ds(off, S_CHUNK)],
            gsem.at[b, sc],
        )

    for pair in range(BATCH // 2):
        b0 = pair * 2
        idx_copies[b0].wait()
        fire_gather(b0, 0)
        idx_copies[b0 + 1].wait()
        fire_gather(b0 + 1, 0)
        for sc in range(1, N_SC):
            fire_gather(b0, sc)
            fire_gather(b0 + 1, sc)
    pos_copy.wait()

    out_copies = []
    for pair in range(BATCH // 2):
        b0 = pair * 2
        for sc in range(N_SC):
            gathers[(b0, sc)].wait()
            gathers[(b0 + 1, sc)].wait()

            def srow(i, carry):
                p = [pos_v[i, pl.ds(k * LANES, LANES)] for k in range(KREG)]
                for b in (b0, b0 + 1):
                    r = b * S_PER_W + i
                    for k in range(KREG):
                        sl = pl.ds(k * LANES, LANES)
                        rows_v[r, sl] = (rows_v[r, sl] + p[k]) * SCALE
                return carry

            lax.fori_loop(sc * S_CHUNK, (sc + 1) * S_CHUNK, srow, 0)

            for b in (b0, b0 + 1):
                out_copies.append(
                    pltpu.async_copy(
                        rows_v.at[pl.ds(b * S_PER_W + sc * S_CHUNK, S_CHUNK)],
                        out_hbm.at[b, pl.ds(s_base + sc * S_CHUNK, S_CHUNK)],
                        osem.at[b, sc],
                    )
                )
    for cp in out_copies:
        cp.wait()


def kernel(x_bs, tok_weight, pos_weight):
    mesh = plsc.VectorSubcoreMesh(core_axis_name="c", subcore_axis_name="s")
    run = functools.partial(
        pl.kernel,
        mesh=mesh,
        out_type=jax.ShapeDtypeStruct((BATCH, SEQ_LEN, EMB), jnp.float32),
        scratch_types=[
            pltpu.VMEM((BATCH, S_PER_W), jnp.int32),
            pltpu.VMEM((ROWS_PER_W, EMB), jnp.float32),
            pltpu.VMEM((S_PER_W, EMB), jnp.float32),
            pltpu.SemaphoreType.DMA((BATCH,)),
            pltpu.SemaphoreType.DMA((BATCH, N_SC)),
            pltpu.SemaphoreType.DMA((BATCH, N_SC)),
            pltpu.SemaphoreType.DMA,
        ],
    )(_body)

    return run(x_bs, tok_weight, pos_weight)
